# Initial kernel scaffold; baseline (speedup 1.0000x reference)
import jax
import jax.numpy as jnp
from jax.experimental import pallas as pl


def kernel(x, edge_index, gamma, beta, W, b):
    raise NotImplementedError("write your pallas kernel here")


# speedup vs baseline: 14.3384x; 14.3384x over previous
import functools

import jax
import jax.numpy as jnp
from jax import lax
from jax.experimental import pallas as pl
from jax.experimental.pallas import tpu as pltpu
from jax.experimental.pallas import tpu_sc as plsc

NC = 2
NS = 16
NW = NC * NS
C = 128
L = 16


def _sc_mesh():
    return plsc.VectorSubcoreMesh(core_axis_name="c", subcore_axis_name="s")


def _make_deg_kernel(R, EP, NCH):
    @functools.partial(
        pl.kernel,
        out_type=jax.ShapeDtypeStruct((NC * R,), jnp.float32),
        mesh=_sc_mesh(),
        scratch_types=[
            pltpu.VMEM((C,), jnp.int32),
            pltpu.VMEM((C,), jnp.float32),
            pltpu.VMEM((R // NS,), jnp.float32),
            pltpu.VMEM_SHARED((R,), jnp.float32),
        ],
    )
    def deg_kernel(dst_hbm, out_hbm, di, ones_v, zv, deg_sh):
        cid = lax.axis_index("c")
        sid = lax.axis_index("s")
        wid = sid * NC + cid
        base = wid * EP
        stripe = R // NS

        ones16 = jnp.ones((L,), jnp.float32)
        zero16 = jnp.zeros((L,), jnp.float32)
        for j in range(C // L):
            ones_v[pl.ds(j * L, L)] = ones16
        for j in range(stripe // L):
            zv[pl.ds(j * L, L)] = zero16
        pltpu.sync_copy(zv, deg_sh.at[pl.ds(sid * stripe, stripe)])
        plsc.subcore_barrier()
        del zero16

        def body(k, _):
            pltpu.sync_copy(dst_hbm.at[pl.ds(base + k * C, C)], di)
            pltpu.sync_copy(ones_v, deg_sh.at[di], add=True)
            return 0

        lax.fori_loop(0, NCH, body, 0)
        plsc.subcore_barrier()
        pltpu.sync_copy(deg_sh.at[pl.ds(sid * stripe, stripe)], zv)
        pltpu.sync_copy(zv, out_hbm.at[pl.ds(cid * R + sid * stripe, stripe)])

    return deg_kernel


def _make_scatter_kernel(N, H, R, EP, NCH):
    @functools.partial(
        pl.kernel,
        out_type=jax.ShapeDtypeStruct((NC, R, H), jnp.float32),
        mesh=_sc_mesh(),
        scratch_types=[
            pltpu.VMEM((C,), jnp.int32),
            pltpu.VMEM((C,), jnp.int32),
            pltpu.VMEM((C, H), jnp.float32),
            pltpu.VMEM((L, H), jnp.float32),
            pltpu.VMEM_SHARED((R, H), jnp.float32),
            pltpu.SemaphoreType.DMA,
        ],
    )
    def scat_kernel(g_hbm, src_hbm, dst_hbm, out_hbm, si, di, rows, zb, acc_sh, sem):
        cid = lax.axis_index("c")
        sid = lax.axis_index("s")
        wid = sid * NC + cid
        base = wid * EP
        stripe = R // NS

        zero16 = jnp.zeros((L,), jnp.float32)
        for i in range(L):
            for j in range(H // L):
                zb[i, pl.ds(j * L, L)] = zero16

        def zbody(r, _):
            pltpu.sync_copy(zb, acc_sh.at[pl.ds(sid * stripe + r * L, L), :])
            return 0

        lax.fori_loop(0, stripe // L, zbody, 0)
        plsc.subcore_barrier()

        def body(k, _):
            pltpu.sync_copy(src_hbm.at[pl.ds(base + k * C, C)], si)
            pltpu.sync_copy(dst_hbm.at[pl.ds(base + k * C, C)], di)
            pltpu.async_copy(g_hbm.at[si], rows, sem).wait()
            pltpu.sync_copy(rows, acc_sh.at[di], add=True)
            return 0

        lax.fori_loop(0, NCH, body, 0)
        plsc.subcore_barrier()

        def obody(r, _):
            row0 = sid * stripe + r * C
            pltpu.sync_copy(acc_sh.at[pl.ds(row0, C), :], rows)
            pltpu.sync_copy(rows, out_hbm.at[cid, pl.ds(row0, C), :])
            return 0

        lax.fori_loop(0, stripe // C, obody, 0)

    return scat_kernel


def _tcab_body(x_ref, gamma_ref, beta_ref, w_ref, deg_ref, g_ref, dinv_ref):
    x = x_ref[...]
    n = x.shape[0]
    mean = jnp.mean(x, axis=0, keepdims=True)
    xc = x - mean
    var = jnp.mean(xc * xc, axis=0, keepdims=True)
    xn = gamma_ref[...] * xc * lax.rsqrt(var + 1e-5) + beta_ref[...]
    deg = deg_ref[0, :n, :] + deg_ref[1, :n, :] + 1.0
    dinv = lax.rsqrt(deg)
    dinv_ref[...] = dinv
    g_ref[...] = jnp.dot(xn * dinv, w_ref[...],
                         preferred_element_type=jnp.float32)


def _tcc_body(acc_ref, g_ref, dinv_ref, b_ref, out_ref):
    s = acc_ref[0] + acc_ref[1] + g_ref[...]
    out_ref[...] = jnp.maximum(dinv_ref[...] * s + b_ref[...], 0.0)


def kernel(x, edge_index, gamma, beta, W, b):
    N, D = x.shape
    H = W.shape[1]
    E = edge_index.shape[1]

    R = ((N + 1 + NS * C - 1) // (NS * C)) * (NS * C)
    Ep = ((E + NW * C - 1) // (NW * C)) * (NW * C)
    EP = Ep // NW
    NCH = EP // C

    src = edge_index[0]
    dst = edge_index[1]
    pad = Ep - E
    if pad:
        src = jnp.concatenate([src, jnp.zeros((pad,), jnp.int32)])
        dst = jnp.concatenate([dst, jnp.full((pad,), N, jnp.int32)])

    deg2 = _make_deg_kernel(R, EP, NCH)(dst)
    deg_col = deg2.reshape(NC, R)[:, :, None]

    g, dinv = pl.pallas_call(
        _tcab_body,
        out_shape=(
            jax.ShapeDtypeStruct((N, H), jnp.float32),
            jax.ShapeDtypeStruct((N, 1), jnp.float32),
        ),
    )(x, gamma, beta, W, deg_col)

    acc2 = _make_scatter_kernel(N, H, R, EP, NCH)(g, src, dst)

    BN = 1000
    grid = (N // BN,)
    out = pl.pallas_call(
        _tcc_body,
        grid=grid,
        in_specs=[
            pl.BlockSpec((NC, BN, H), lambda i: (0, i, 0)),
            pl.BlockSpec((BN, H), lambda i: (i, 0)),
            pl.BlockSpec((BN, 1), lambda i: (i, 0)),
            pl.BlockSpec((H,), lambda i: (0,)),
        ],
        out_specs=pl.BlockSpec((BN, H), lambda i: (i, 0)),
        out_shape=jax.ShapeDtypeStruct((N, H), jnp.float32),
    )(acc2, g, dinv, b)
    return out
